# baseline (speedup 1.0000x reference)
import functools

import jax
import jax.numpy as jnp
from jax import lax
from jax.experimental import pallas as pl
from jax.experimental.pallas import tpu as pltpu
from jax.experimental.pallas import tpu_sc as plsc

LANES = 16
CHUNK = 128
NBUF = 4
ET = 8


def _build(batch, seq, vocab, embed, n_workers):
    total_rows = batch * seq
    per_w = total_rows // n_workers
    n_chunks = per_w // CHUNK
    n_groups = n_chunks // NBUF
    blocks_per_s = batch // CHUNK
    n_et = embed // ET
    seg = ET * CHUNK
    mesh = plsc.VectorSubcoreMesh(core_axis_name="c", subcore_axis_name="s")
    num_cores = 2

    @functools.partial(
        pl.kernel,
        mesh=mesh,
        compiler_params=pltpu.CompilerParams(use_tc_tiling_on_sc=False,
                                             needs_layout_passes=False),
        out_type=jax.ShapeDtypeStruct((total_rows * embed,), jnp.float32),
        scratch_types=[
            pltpu.VMEM((n_chunks, CHUNK), jnp.int32),
            pltpu.VMEM((NBUF, CHUNK, embed), jnp.float32),
            pltpu.VMEM((CHUNK * embed,), jnp.float32),
            pltpu.VMEM((CHUNK * embed,), jnp.float32),
            pltpu.VMEM((CHUNK * embed,), jnp.float32),
            pltpu.VMEM((CHUNK * embed,), jnp.float32),
            pltpu.VMEM((seq, embed), jnp.float32),
            pltpu.SemaphoreType.DMA,
            pltpu.SemaphoreType.DMA,
            pltpu.SemaphoreType.DMA,
            pltpu.SemaphoreType.DMA,
            pltpu.SemaphoreType.DMA,
            pltpu.SemaphoreType.DMA,
            pltpu.SemaphoreType.DMA,
            pltpu.SemaphoreType.DMA,
        ],
    )
    def emb_kernel(idx_hbm, table_hbm, pos_hbm, out_hbm,
                   idx_all, rows_v, t0, t1, t2, t3, pos_v, *sems):
        rowst = (t0, t1, t2, t3)
        semg = sems[:NBUF]
        semw = sems[NBUF:]
        wid = lax.axis_index("s") * num_cores + lax.axis_index("c")
        f0 = wid * n_chunks

        pltpu.sync_copy(pos_hbm, pos_v)
        pltpu.sync_copy(idx_hbm.at[pl.ds(f0, n_chunks)], idx_all)

        iota16 = lax.iota(jnp.int32, LANES)

        def fire(c, b):
            pltpu.async_copy(table_hbm.at[idx_all.at[c]], rows_v.at[b],
                             semg[b])

        def drain_g(b):
            pltpu.make_async_copy(table_hbm.at[idx_all.at[0]], rows_v.at[b],
                                  semg[b]).wait()

        def drain_w(b):
            pltpu.make_async_copy(rowst[b],
                                  out_hbm.at[pl.ds(0, CHUNK * embed)],
                                  semw[b]).wait()

        def transpose_add(s, b):
            sb = jnp.broadcast_to(s, (LANES,))

            def d_body(d0, _):
                for du in range(4):
                    d = d0 * 4 + du
                    for eb in range(embed // LANES):
                        ce = eb * LANES + lax.rem(iota16 + d, LANES)
                        pe = plsc.load_gather(pos_v, [sb, ce])
                        cd = ce * CHUNK
                        for rb in range(CHUNK // LANES):
                            ridx = iota16 + rb * LANES
                            v = plsc.load_gather(rows_v.at[b], [ridx, ce])
                            plsc.store_scatter(rowst[b], [cd + ridx], v + pe)
                return _

            lax.fori_loop(0, LANES // 4, d_body, None)

        def step(c, b, wait_w, fire_ahead):
            drain_g(b)
            s = c // blocks_per_s
            bt = lax.rem(c, blocks_per_s)
            transpose_add(s, b)
            obase = s * (embed * batch) + bt * (ET * CHUNK)
            for et in range(n_et):
                pltpu.async_copy(
                    rowst[b].at[pl.ds(et * seg, seg)],
                    out_hbm.at[pl.ds(obase + et * (blocks_per_s * seg), seg)],
                    semw[b])
            bf = (b + NBUF - 1) % NBUF
            if wait_w:
                drain_w(bf)
            if fire_ahead:
                fire(c - f0 + NBUF - 1, bf)

        for b in range(NBUF - 1):
            fire(b, b)
        for b in range(NBUF):
            step(f0 + b, b, wait_w=(b > 0), fire_ahead=True)

        def group_body(g, _):
            c0 = f0 + g * NBUF
            for b in range(NBUF):
                step(c0 + b, b, wait_w=True, fire_ahead=True)
            return _

        lax.fori_loop(1, n_groups - 1, group_body, None)

        cL = f0 + (n_groups - 1) * NBUF
        step(cL, 0, wait_w=True, fire_ahead=True)
        for b in range(1, NBUF):
            step(cL + b, b, wait_w=True, fire_ahead=False)
        drain_w(NBUF - 1)

    return emb_kernel


def _build_detranspose(vocab, embed, n_workers):
    n_blocks = vocab // CHUNK
    vmain = n_blocks * CHUNK
    tail = vocab - vmain
    per_w = n_blocks // n_workers
    n_extra = n_blocks - per_w * n_workers
    mesh = plsc.VectorSubcoreMesh(core_axis_name="c", subcore_axis_name="s")
    num_cores = 2

    @functools.partial(
        pl.kernel,
        mesh=mesh,
        compiler_params=pltpu.CompilerParams(use_tc_tiling_on_sc=True,
                                             needs_layout_passes=False),
        out_type=jax.ShapeDtypeStruct((vocab * embed,), jnp.float32),
        scratch_types=(
            [pltpu.VMEM((embed, CHUNK), jnp.float32)] * NBUF
            + [pltpu.VMEM((CHUNK * embed,), jnp.float32)] * NBUF
            + [pltpu.VMEM((tail * embed,), jnp.float32)]
            + [pltpu.SemaphoreType.DMA] * (2 * NBUF)
        ),
    )
    def det_kernel(tt_hbm, tail_hbm, out_hbm, *rest):
        inb = rest[:NBUF]
        outb = rest[NBUF:2 * NBUF]
        tail_v = rest[2 * NBUF]
        semi = rest[2 * NBUF + 1:3 * NBUF + 1]
        semo = rest[3 * NBUF + 1:]
        wid = lax.axis_index("s") * num_cores + lax.axis_index("c")
        blk0 = wid * per_w + jnp.minimum(wid, n_extra)

        iota16 = lax.iota(jnp.int32, LANES)

        def fire_i(i, p):
            pltpu.async_copy(
                tt_hbm.at[:, pl.ds((blk0 + i) * CHUNK, CHUNK)],
                inb[p], semi[p])

        def drain_i(p):
            pltpu.make_async_copy(tt_hbm.at[:, pl.ds(0, CHUNK)],
                                  inb[p], semi[p]).wait()

        def fire_o(i, p):
            pltpu.async_copy(outb[p],
                             out_hbm.at[pl.ds((blk0 + i) * (CHUNK * embed),
                                              CHUNK * embed)],
                             semo[p])

        def drain_o(p):
            pltpu.make_async_copy(outb[p],
                                  out_hbm.at[pl.ds(0, CHUNK * embed)],
                                  semo[p]).wait()

        def transpose_blk(p):
            def d_body(d, _):
                for eb in range(embed // LANES):
                    ce = eb * LANES + lax.rem(iota16 + d, LANES)
                    for vb in range(CHUNK // LANES):
                        vidx = iota16 + vb * LANES
                        v = plsc.load_gather(inb[p], [ce, vidx])
                        plsc.store_scatter(outb[p], [vidx * embed + ce], v)
                return _

            lax.fori_loop(0, LANES, d_body, None)

        def step(i, p, wait_o, fire_next):
            drain_i(p)
            transpose_blk(p)
            fire_o(i, p)
            pf = (p + NBUF - 1) % NBUF
            if wait_o:
                drain_o(pf)
            if fire_next:
                fire_i(i + NBUF - 1, pf)

        n_det_groups = per_w // NBUF
        for p in range(NBUF - 1):
            fire_i(p, p)
        for p in range(NBUF):
            step(p, p, wait_o=(p > 0), fire_next=True)

        def group_body(g, _):
            s0 = g * NBUF
            for p in range(NBUF):
                step(s0 + p, p, wait_o=True, fire_next=True)
            return _

        lax.fori_loop(1, n_det_groups - 1, group_body, None)

        sL = (n_det_groups - 1) * NBUF
        step(sL, 0, wait_o=True, fire_next=True)
        for p in range(1, NBUF):
            step(sL + p, p, wait_o=True, fire_next=False)
        drain_o(NBUF - 1)

        @pl.when(wid < n_extra)
        def _extra():
            pltpu.sync_copy(
                tt_hbm.at[:, pl.ds((blk0 + per_w) * CHUNK, CHUNK)], inb[0])
            transpose_blk(0)
            pltpu.sync_copy(outb[0],
                            out_hbm.at[pl.ds((blk0 + per_w) * (CHUNK * embed),
                                             CHUNK * embed)])

        @pl.when(wid == n_workers - 1)
        def _tail():
            pltpu.sync_copy(tail_hbm, tail_v)
            pltpu.sync_copy(tail_v,
                            out_hbm.at[pl.ds(vmain * embed, tail * embed)])

    return det_kernel


def _build_tc_detranspose(vocab, embed):
    bw = 512
    grid = -(-(vocab // 4) // bw)
    q_pad = grid * bw

    def body(t0, t1, t2, t3, out_ref):
        for j, ref in enumerate((t0, t1, t2, t3)):
            out_ref[:, pl.ds(j * embed, embed)] = ref[...].T

    in_specs = [
        pl.BlockSpec((embed, bw),
                     functools.partial(lambda j, i: (0, j * grid + i), j))
        for j in range(3)
    ] + [pl.BlockSpec((embed, bw), lambda i: (0, i))]
    fn = pl.pallas_call(
        body,
        grid=(grid,),
        in_specs=in_specs,
        out_specs=pl.BlockSpec((bw, 4 * embed), lambda i: (i, 0)),
        out_shape=jax.ShapeDtypeStruct((q_pad, 4 * embed), jnp.float32),
    )
    return fn, q_pad


def kernel(inputs, word_table, pos_table):
    batch, seq = inputs.shape
    vocab, embed = word_table.shape
    n_workers = 32

    det, q_pad = _build_tc_detranspose(vocab, embed)
    tt = word_table.T
    tt_last = jnp.pad(lax.slice(tt, (0, 3 * q_pad), (embed, vocab)),
                      ((0, 0), (0, 4 * q_pad - vocab)))
    table_packed = det(tt, tt, tt, tt_last).reshape(4 * q_pad, embed)

    idx = inputs.T.reshape(batch * seq // CHUNK, CHUNK).astype(jnp.int32)
    idx = (idx % q_pad) * 4 + idx // q_pad
    fn = _build(batch, seq, vocab, embed, n_workers)
    flat = fn(idx, table_packed, pos_table)
    x = flat.reshape(seq, embed // ET, batch // CHUNK, ET, CHUNK)
    return x.transpose(2, 4, 0, 1, 3).reshape(batch, seq, embed)


# speedup vs baseline: 1.1409x; 1.1409x over previous
import functools

import jax
import jax.numpy as jnp
from jax import lax
from jax.experimental import pallas as pl
from jax.experimental.pallas import tpu as pltpu
from jax.experimental.pallas import tpu_sc as plsc

LANES = 16
CHUNK = 128
NBUF = 4
ET = 8


def _build(batch, seq, vocab, embed, n_workers):
    total_rows = batch * seq
    per_w = total_rows // n_workers
    n_chunks = per_w // CHUNK
    n_groups = n_chunks // NBUF
    blocks_per_s = batch // CHUNK
    n_et = embed // ET
    seg = ET * CHUNK
    mesh = plsc.VectorSubcoreMesh(core_axis_name="c", subcore_axis_name="s")
    num_cores = 2

    @functools.partial(
        pl.kernel,
        mesh=mesh,
        compiler_params=pltpu.CompilerParams(use_tc_tiling_on_sc=False,
                                             needs_layout_passes=False),
        out_type=jax.ShapeDtypeStruct((total_rows * embed,), jnp.float32),
        scratch_types=[
            pltpu.VMEM((n_chunks, CHUNK), jnp.int32),
            pltpu.VMEM((NBUF, CHUNK, embed), jnp.float32),
            pltpu.VMEM((CHUNK * embed,), jnp.float32),
            pltpu.VMEM((CHUNK * embed,), jnp.float32),
            pltpu.VMEM((CHUNK * embed,), jnp.float32),
            pltpu.VMEM((CHUNK * embed,), jnp.float32),
            pltpu.VMEM((seq, embed), jnp.float32),
            pltpu.SemaphoreType.DMA,
            pltpu.SemaphoreType.DMA,
            pltpu.SemaphoreType.DMA,
            pltpu.SemaphoreType.DMA,
            pltpu.SemaphoreType.DMA,
            pltpu.SemaphoreType.DMA,
            pltpu.SemaphoreType.DMA,
            pltpu.SemaphoreType.DMA,
        ],
    )
    def emb_kernel(idx_hbm, table_hbm, pos_hbm, out_hbm,
                   idx_all, rows_v, t0, t1, t2, t3, pos_v, *sems):
        rowst = (t0, t1, t2, t3)
        semg = sems[:NBUF]
        semw = sems[NBUF:]
        wid = lax.axis_index("s") * num_cores + lax.axis_index("c")
        f0 = wid * n_chunks

        pltpu.sync_copy(pos_hbm, pos_v)
        pltpu.sync_copy(idx_hbm.at[pl.ds(f0, n_chunks)], idx_all)

        iota16 = lax.iota(jnp.int32, LANES)

        def fire(c, b):
            pltpu.async_copy(table_hbm.at[idx_all.at[c]], rows_v.at[b],
                             semg[b])

        def drain_g(b):
            pltpu.make_async_copy(table_hbm.at[idx_all.at[0]], rows_v.at[b],
                                  semg[b]).wait()

        def drain_w(b):
            pltpu.make_async_copy(rowst[b],
                                  out_hbm.at[pl.ds(0, CHUNK * embed)],
                                  semw[b]).wait()

        def transpose_add(s, b):
            sb = jnp.broadcast_to(s, (LANES,))

            def d_body(d0, _):
                for du in range(4):
                    d = d0 * 4 + du
                    for eb in range(embed // LANES):
                        ce = eb * LANES + lax.rem(iota16 + d, LANES)
                        pe = plsc.load_gather(pos_v, [sb, ce])
                        cd = ce * CHUNK
                        for rb in range(CHUNK // LANES):
                            ridx = iota16 + rb * LANES
                            v = plsc.load_gather(rows_v.at[b], [ridx, ce])
                            plsc.store_scatter(rowst[b], [cd + ridx], v + pe)
                return _

            lax.fori_loop(0, LANES // 4, d_body, None)

        def step(c, b, wait_w, fire_ahead):
            drain_g(b)
            s = c // blocks_per_s
            bt = lax.rem(c, blocks_per_s)
            transpose_add(s, b)
            obase = s * (embed * batch) + bt * (ET * CHUNK)
            for et in range(n_et):
                pltpu.async_copy(
                    rowst[b].at[pl.ds(et * seg, seg)],
                    out_hbm.at[pl.ds(obase + et * (blocks_per_s * seg), seg)],
                    semw[b])
            bf = (b + NBUF - 1) % NBUF
            if wait_w:
                drain_w(bf)
            if fire_ahead:
                fire(c - f0 + NBUF - 1, bf)

        for b in range(NBUF - 1):
            fire(b, b)
        for b in range(NBUF):
            step(f0 + b, b, wait_w=(b > 0), fire_ahead=True)

        def group_body(g, _):
            c0 = f0 + g * NBUF
            for b in range(NBUF):
                step(c0 + b, b, wait_w=True, fire_ahead=True)
            return _

        lax.fori_loop(1, n_groups - 1, group_body, None)

        cL = f0 + (n_groups - 1) * NBUF
        step(cL, 0, wait_w=True, fire_ahead=True)
        for b in range(1, NBUF):
            step(cL + b, b, wait_w=True, fire_ahead=False)
        drain_w(NBUF - 1)

    return emb_kernel


def _build_detranspose(vocab, embed, n_workers):
    n_blocks = vocab // CHUNK
    vmain = n_blocks * CHUNK
    tail = vocab - vmain
    per_w = n_blocks // n_workers
    n_extra = n_blocks - per_w * n_workers
    mesh = plsc.VectorSubcoreMesh(core_axis_name="c", subcore_axis_name="s")
    num_cores = 2

    @functools.partial(
        pl.kernel,
        mesh=mesh,
        compiler_params=pltpu.CompilerParams(use_tc_tiling_on_sc=True,
                                             needs_layout_passes=False),
        out_type=jax.ShapeDtypeStruct((vocab * embed,), jnp.float32),
        scratch_types=(
            [pltpu.VMEM((embed, CHUNK), jnp.float32)] * NBUF
            + [pltpu.VMEM((CHUNK * embed,), jnp.float32)] * NBUF
            + [pltpu.VMEM((tail * embed,), jnp.float32)]
            + [pltpu.SemaphoreType.DMA] * (2 * NBUF)
        ),
    )
    def det_kernel(tt_hbm, tail_hbm, out_hbm, *rest):
        inb = rest[:NBUF]
        outb = rest[NBUF:2 * NBUF]
        tail_v = rest[2 * NBUF]
        semi = rest[2 * NBUF + 1:3 * NBUF + 1]
        semo = rest[3 * NBUF + 1:]
        wid = lax.axis_index("s") * num_cores + lax.axis_index("c")
        blk0 = wid * per_w + jnp.minimum(wid, n_extra)

        iota16 = lax.iota(jnp.int32, LANES)

        def fire_i(i, p):
            pltpu.async_copy(
                tt_hbm.at[:, pl.ds((blk0 + i) * CHUNK, CHUNK)],
                inb[p], semi[p])

        def drain_i(p):
            pltpu.make_async_copy(tt_hbm.at[:, pl.ds(0, CHUNK)],
                                  inb[p], semi[p]).wait()

        def fire_o(i, p):
            pltpu.async_copy(outb[p],
                             out_hbm.at[pl.ds((blk0 + i) * (CHUNK * embed),
                                              CHUNK * embed)],
                             semo[p])

        def drain_o(p):
            pltpu.make_async_copy(outb[p],
                                  out_hbm.at[pl.ds(0, CHUNK * embed)],
                                  semo[p]).wait()

        def transpose_blk(p):
            def d_body(d, _):
                for eb in range(embed // LANES):
                    ce = eb * LANES + lax.rem(iota16 + d, LANES)
                    for vb in range(CHUNK // LANES):
                        vidx = iota16 + vb * LANES
                        v = plsc.load_gather(inb[p], [ce, vidx])
                        plsc.store_scatter(outb[p], [vidx * embed + ce], v)
                return _

            lax.fori_loop(0, LANES, d_body, None)

        def step(i, p, wait_o, fire_next):
            drain_i(p)
            transpose_blk(p)
            fire_o(i, p)
            pf = (p + NBUF - 1) % NBUF
            if wait_o:
                drain_o(pf)
            if fire_next:
                fire_i(i + NBUF - 1, pf)

        n_det_groups = per_w // NBUF
        for p in range(NBUF - 1):
            fire_i(p, p)
        for p in range(NBUF):
            step(p, p, wait_o=(p > 0), fire_next=True)

        def group_body(g, _):
            s0 = g * NBUF
            for p in range(NBUF):
                step(s0 + p, p, wait_o=True, fire_next=True)
            return _

        lax.fori_loop(1, n_det_groups - 1, group_body, None)

        sL = (n_det_groups - 1) * NBUF
        step(sL, 0, wait_o=True, fire_next=True)
        for p in range(1, NBUF):
            step(sL + p, p, wait_o=True, fire_next=False)
        drain_o(NBUF - 1)

        @pl.when(wid < n_extra)
        def _extra():
            pltpu.sync_copy(
                tt_hbm.at[:, pl.ds((blk0 + per_w) * CHUNK, CHUNK)], inb[0])
            transpose_blk(0)
            pltpu.sync_copy(outb[0],
                            out_hbm.at[pl.ds((blk0 + per_w) * (CHUNK * embed),
                                             CHUNK * embed)])

        @pl.when(wid == n_workers - 1)
        def _tail():
            pltpu.sync_copy(tail_hbm, tail_v)
            pltpu.sync_copy(tail_v,
                            out_hbm.at[pl.ds(vmain * embed, tail * embed)])

    return det_kernel


def _build_tc_detranspose(vocab, embed):
    bw = 512
    grid = -(-(vocab // 4) // bw)
    q_pad = grid * bw

    def body(t0, t1, t2, t3, out_ref):
        x = jnp.concatenate(
            [t0[...], t1[...], t2[...], t3[...]], axis=0)
        out_ref[...] = x.T

    in_specs = [
        pl.BlockSpec((embed, bw),
                     functools.partial(lambda j, i: (0, j * grid + i), j))
        for j in range(3)
    ] + [pl.BlockSpec((embed, bw), lambda i: (0, i))]
    fn = pl.pallas_call(
        body,
        grid=(grid,),
        in_specs=in_specs,
        out_specs=pl.BlockSpec((bw, 4 * embed), lambda i: (i, 0)),
        out_shape=jax.ShapeDtypeStruct((q_pad, 4 * embed), jnp.float32),
    )
    return fn, q_pad


def kernel(inputs, word_table, pos_table):
    batch, seq = inputs.shape
    vocab, embed = word_table.shape
    n_workers = 32

    det, q_pad = _build_tc_detranspose(vocab, embed)
    tt = word_table.T
    tt_last = jnp.pad(lax.slice(tt, (0, 3 * q_pad), (embed, vocab)),
                      ((0, 0), (0, 4 * q_pad - vocab)))
    table_packed = det(tt, tt, tt, tt_last).reshape(4 * q_pad, embed)

    idx = inputs.T.reshape(batch * seq // CHUNK, CHUNK).astype(jnp.int32)
    idx = (idx % q_pad) * 4 + idx // q_pad
    fn = _build(batch, seq, vocab, embed, n_workers)
    flat = fn(idx, table_packed, pos_table)
    x = flat.reshape(seq, embed // ET, batch // CHUNK, ET, CHUNK)
    return x.transpose(2, 4, 0, 1, 3).reshape(batch, seq, embed)


# speedup vs baseline: 1.5896x; 1.3933x over previous
import functools

import jax
import jax.numpy as jnp
from jax import lax
from jax.experimental import pallas as pl
from jax.experimental.pallas import tpu as pltpu
from jax.experimental.pallas import tpu_sc as plsc

LANES = 16
CHUNK = 128
NBUF = 4
ET = 8


def _build(batch, seq, vocab, embed, n_workers):
    total_rows = batch * seq
    per_w = total_rows // n_workers
    n_chunks = per_w // CHUNK
    n_groups = n_chunks // NBUF
    blocks_per_s = batch // CHUNK
    n_et = embed // ET
    seg = ET * CHUNK
    mesh = plsc.VectorSubcoreMesh(core_axis_name="c", subcore_axis_name="s")
    num_cores = 2

    @functools.partial(
        pl.kernel,
        mesh=mesh,
        compiler_params=pltpu.CompilerParams(use_tc_tiling_on_sc=False,
                                             needs_layout_passes=False),
        out_type=jax.ShapeDtypeStruct((total_rows * embed,), jnp.float32),
        scratch_types=[
            pltpu.VMEM((n_chunks, CHUNK), jnp.int32),
            pltpu.VMEM((NBUF, CHUNK, embed), jnp.float32),
            pltpu.VMEM((CHUNK * embed,), jnp.float32),
            pltpu.VMEM((CHUNK * embed,), jnp.float32),
            pltpu.VMEM((CHUNK * embed,), jnp.float32),
            pltpu.VMEM((CHUNK * embed,), jnp.float32),
            pltpu.VMEM((seq, embed), jnp.float32),
            pltpu.SemaphoreType.DMA,
            pltpu.SemaphoreType.DMA,
            pltpu.SemaphoreType.DMA,
            pltpu.SemaphoreType.DMA,
            pltpu.SemaphoreType.DMA,
            pltpu.SemaphoreType.DMA,
            pltpu.SemaphoreType.DMA,
            pltpu.SemaphoreType.DMA,
        ],
    )
    def emb_kernel(idx_hbm, table_hbm, pos_hbm, out_hbm,
                   idx_all, rows_v, t0, t1, t2, t3, pos_v, *sems):
        rowst = (t0, t1, t2, t3)
        semg = sems[:NBUF]
        semw = sems[NBUF:]
        wid = lax.axis_index("s") * num_cores + lax.axis_index("c")
        f0 = wid * n_chunks

        pltpu.sync_copy(pos_hbm, pos_v)
        pltpu.sync_copy(idx_hbm.at[pl.ds(f0, n_chunks)], idx_all)

        iota16 = lax.iota(jnp.int32, LANES)

        def fire(c, b):
            pltpu.async_copy(table_hbm.at[idx_all.at[c]], rows_v.at[b],
                             semg[b])

        def drain_g(b):
            pltpu.make_async_copy(table_hbm.at[idx_all.at[0]], rows_v.at[b],
                                  semg[b]).wait()

        def drain_w(b):
            pltpu.make_async_copy(rowst[b],
                                  out_hbm.at[pl.ds(0, CHUNK * embed)],
                                  semw[b]).wait()

        def transpose_add(s, b):
            sb = jnp.broadcast_to(s, (LANES,))

            def d_body(d0, _):
                for du in range(4):
                    d = d0 * 4 + du
                    for eb in range(embed // LANES):
                        ce = eb * LANES + lax.rem(iota16 + d, LANES)
                        pe = plsc.load_gather(pos_v, [sb, ce])
                        cd = ce * CHUNK
                        for rb in range(CHUNK // LANES):
                            ridx = iota16 + rb * LANES
                            v = plsc.load_gather(rows_v.at[b], [ridx, ce])
                            plsc.store_scatter(rowst[b], [cd + ridx], v + pe)
                return _

            lax.fori_loop(0, LANES // 4, d_body, None)

        def step(c, b, wait_w, fire_ahead):
            drain_g(b)
            s = c // blocks_per_s
            bt = lax.rem(c, blocks_per_s)
            transpose_add(s, b)
            obase = s * (embed * batch) + bt * (ET * CHUNK)
            for et in range(n_et):
                pltpu.async_copy(
                    rowst[b].at[pl.ds(et * seg, seg)],
                    out_hbm.at[pl.ds(obase + et * (blocks_per_s * seg), seg)],
                    semw[b])
            bf = (b + NBUF - 1) % NBUF
            if wait_w:
                drain_w(bf)
            if fire_ahead:
                fire(c - f0 + NBUF - 1, bf)

        for b in range(NBUF - 1):
            fire(b, b)
        for b in range(NBUF):
            step(f0 + b, b, wait_w=(b > 0), fire_ahead=True)

        def group_body(g, _):
            c0 = f0 + g * NBUF
            for b in range(NBUF):
                step(c0 + b, b, wait_w=True, fire_ahead=True)
            return _

        lax.fori_loop(1, n_groups - 1, group_body, None)

        cL = f0 + (n_groups - 1) * NBUF
        step(cL, 0, wait_w=True, fire_ahead=True)
        for b in range(1, NBUF):
            step(cL + b, b, wait_w=True, fire_ahead=False)
        drain_w(NBUF - 1)

    return emb_kernel


def _build_detranspose(vocab, embed, n_workers):
    n_blocks = vocab // CHUNK
    vmain = n_blocks * CHUNK
    tail = vocab - vmain
    per_w = n_blocks // n_workers
    n_extra = n_blocks - per_w * n_workers
    mesh = plsc.VectorSubcoreMesh(core_axis_name="c", subcore_axis_name="s")
    num_cores = 2

    @functools.partial(
        pl.kernel,
        mesh=mesh,
        compiler_params=pltpu.CompilerParams(use_tc_tiling_on_sc=True,
                                             needs_layout_passes=False),
        out_type=jax.ShapeDtypeStruct((vocab * embed,), jnp.float32),
        scratch_types=(
            [pltpu.VMEM((embed, CHUNK), jnp.float32)] * NBUF
            + [pltpu.VMEM((CHUNK * embed,), jnp.float32)] * NBUF
            + [pltpu.VMEM((tail * embed,), jnp.float32)]
            + [pltpu.SemaphoreType.DMA] * (2 * NBUF)
        ),
    )
    def det_kernel(tt_hbm, tail_hbm, out_hbm, *rest):
        inb = rest[:NBUF]
        outb = rest[NBUF:2 * NBUF]
        tail_v = rest[2 * NBUF]
        semi = rest[2 * NBUF + 1:3 * NBUF + 1]
        semo = rest[3 * NBUF + 1:]
        wid = lax.axis_index("s") * num_cores + lax.axis_index("c")
        blk0 = wid * per_w + jnp.minimum(wid, n_extra)

        iota16 = lax.iota(jnp.int32, LANES)

        def fire_i(i, p):
            pltpu.async_copy(
                tt_hbm.at[:, pl.ds((blk0 + i) * CHUNK, CHUNK)],
                inb[p], semi[p])

        def drain_i(p):
            pltpu.make_async_copy(tt_hbm.at[:, pl.ds(0, CHUNK)],
                                  inb[p], semi[p]).wait()

        def fire_o(i, p):
            pltpu.async_copy(outb[p],
                             out_hbm.at[pl.ds((blk0 + i) * (CHUNK * embed),
                                              CHUNK * embed)],
                             semo[p])

        def drain_o(p):
            pltpu.make_async_copy(outb[p],
                                  out_hbm.at[pl.ds(0, CHUNK * embed)],
                                  semo[p]).wait()

        def transpose_blk(p):
            def d_body(d, _):
                for eb in range(embed // LANES):
                    ce = eb * LANES + lax.rem(iota16 + d, LANES)
                    for vb in range(CHUNK // LANES):
                        vidx = iota16 + vb * LANES
                        v = plsc.load_gather(inb[p], [ce, vidx])
                        plsc.store_scatter(outb[p], [vidx * embed + ce], v)
                return _

            lax.fori_loop(0, LANES, d_body, None)

        def step(i, p, wait_o, fire_next):
            drain_i(p)
            transpose_blk(p)
            fire_o(i, p)
            pf = (p + NBUF - 1) % NBUF
            if wait_o:
                drain_o(pf)
            if fire_next:
                fire_i(i + NBUF - 1, pf)

        n_det_groups = per_w // NBUF
        for p in range(NBUF - 1):
            fire_i(p, p)
        for p in range(NBUF):
            step(p, p, wait_o=(p > 0), fire_next=True)

        def group_body(g, _):
            s0 = g * NBUF
            for p in range(NBUF):
                step(s0 + p, p, wait_o=True, fire_next=True)
            return _

        lax.fori_loop(1, n_det_groups - 1, group_body, None)

        sL = (n_det_groups - 1) * NBUF
        step(sL, 0, wait_o=True, fire_next=True)
        for p in range(1, NBUF):
            step(sL + p, p, wait_o=True, fire_next=False)
        drain_o(NBUF - 1)

        @pl.when(wid < n_extra)
        def _extra():
            pltpu.sync_copy(
                tt_hbm.at[:, pl.ds((blk0 + per_w) * CHUNK, CHUNK)], inb[0])
            transpose_blk(0)
            pltpu.sync_copy(outb[0],
                            out_hbm.at[pl.ds((blk0 + per_w) * (CHUNK * embed),
                                             CHUNK * embed)])

        @pl.when(wid == n_workers - 1)
        def _tail():
            pltpu.sync_copy(tail_hbm, tail_v)
            pltpu.sync_copy(tail_v,
                            out_hbm.at[pl.ds(vmain * embed, tail * embed)])

    return det_kernel


def _build_tc_detranspose(vocab, embed):
    bw = 2048
    grid = -(-(vocab // 4) // bw)
    q_pad = grid * bw

    def body(t0, t1, t2, t3, out_ref):
        x = jnp.concatenate(
            [t0[...], t1[...], t2[...], t3[...]], axis=0)
        out_ref[...] = x.T

    in_specs = [
        pl.BlockSpec((embed, bw),
                     functools.partial(lambda j, i: (0, j * grid + i), j))
        for j in range(3)
    ] + [pl.BlockSpec((embed, bw), lambda i: (0, i))]
    fn = pl.pallas_call(
        body,
        grid=(grid,),
        in_specs=in_specs,
        out_specs=pl.BlockSpec((bw, 4 * embed), lambda i: (i, 0)),
        out_shape=jax.ShapeDtypeStruct((q_pad, 4 * embed), jnp.float32),
    )
    return fn, q_pad


def kernel(inputs, word_table, pos_table):
    batch, seq = inputs.shape
    vocab, embed = word_table.shape
    n_workers = 32

    det, q_pad = _build_tc_detranspose(vocab, embed)
    tt = word_table.T
    tt_last = jnp.pad(lax.slice(tt, (0, 3 * q_pad), (embed, vocab)),
                      ((0, 0), (0, 4 * q_pad - vocab)))
    table_packed = det(tt, tt, tt, tt_last).reshape(4 * q_pad, embed)

    idx = inputs.T.reshape(batch * seq // CHUNK, CHUNK).astype(jnp.int32)
    idx = (idx % q_pad) * 4 + idx // q_pad
    fn = _build(batch, seq, vocab, embed, n_workers)
    flat = fn(idx, table_packed, pos_table)
    x = flat.reshape(seq, embed // ET, batch // CHUNK, ET, CHUNK)
    return x.transpose(2, 4, 0, 1, 3).reshape(batch, seq, embed)


# speedup vs baseline: 1.7187x; 1.0812x over previous
import functools

import jax
import jax.numpy as jnp
from jax import lax
from jax.experimental import pallas as pl
from jax.experimental.pallas import tpu as pltpu
from jax.experimental.pallas import tpu_sc as plsc

LANES = 16
CHUNK = 128
NBUF = 4
ET = 8


def _build(batch, seq, vocab, embed, n_workers):
    total_rows = batch * seq
    per_w = total_rows // n_workers
    n_chunks = per_w // CHUNK
    n_groups = n_chunks // NBUF
    blocks_per_s = batch // CHUNK
    n_et = embed // ET
    seg = ET * CHUNK
    mesh = plsc.VectorSubcoreMesh(core_axis_name="c", subcore_axis_name="s")
    num_cores = 2

    @functools.partial(
        pl.kernel,
        mesh=mesh,
        compiler_params=pltpu.CompilerParams(use_tc_tiling_on_sc=False,
                                             needs_layout_passes=False),
        out_type=jax.ShapeDtypeStruct((total_rows * embed,), jnp.float32),
        scratch_types=[
            pltpu.VMEM((n_chunks, CHUNK), jnp.int32),
            pltpu.VMEM((NBUF, CHUNK, embed), jnp.float32),
            pltpu.VMEM((CHUNK * embed,), jnp.float32),
            pltpu.VMEM((CHUNK * embed,), jnp.float32),
            pltpu.VMEM((CHUNK * embed,), jnp.float32),
            pltpu.VMEM((CHUNK * embed,), jnp.float32),
            pltpu.VMEM((seq, embed), jnp.float32),
            pltpu.SemaphoreType.DMA,
            pltpu.SemaphoreType.DMA,
            pltpu.SemaphoreType.DMA,
            pltpu.SemaphoreType.DMA,
            pltpu.SemaphoreType.DMA,
            pltpu.SemaphoreType.DMA,
            pltpu.SemaphoreType.DMA,
            pltpu.SemaphoreType.DMA,
        ],
    )
    def emb_kernel(idx_hbm, table_hbm, pos_hbm, out_hbm,
                   idx_all, rows_v, t0, t1, t2, t3, pos_v, *sems):
        rowst = (t0, t1, t2, t3)
        semg = sems[:NBUF]
        semw = sems[NBUF:]
        wid = lax.axis_index("s") * num_cores + lax.axis_index("c")
        f0 = wid * n_chunks

        pltpu.sync_copy(pos_hbm, pos_v)
        pltpu.sync_copy(idx_hbm.at[pl.ds(f0, n_chunks)], idx_all)

        iota16 = lax.iota(jnp.int32, LANES)

        def fire(c, b):
            pltpu.async_copy(table_hbm.at[idx_all.at[c]], rows_v.at[b],
                             semg[b])

        def drain_g(b):
            pltpu.make_async_copy(table_hbm.at[idx_all.at[0]], rows_v.at[b],
                                  semg[b]).wait()

        def drain_w(b):
            pltpu.make_async_copy(rowst[b],
                                  out_hbm.at[pl.ds(0, CHUNK * embed)],
                                  semw[b]).wait()

        def transpose_add(s, b):
            sb = jnp.broadcast_to(s, (LANES,))

            def d_body(d0, _):
                for du in range(4):
                    d = d0 * 4 + du
                    for eb in range(embed // LANES):
                        ce = eb * LANES + lax.rem(iota16 + d, LANES)
                        pe = plsc.load_gather(pos_v, [sb, ce])
                        cd = ce * CHUNK
                        for rb in range(CHUNK // LANES):
                            ridx = iota16 + rb * LANES
                            v = plsc.load_gather(rows_v.at[b], [ridx, ce])
                            plsc.store_scatter(rowst[b], [cd + ridx], v + pe)
                return _

            lax.fori_loop(0, LANES // 4, d_body, None)

        def step(c, b, wait_w, fire_ahead):
            drain_g(b)
            s = c // blocks_per_s
            bt = lax.rem(c, blocks_per_s)
            transpose_add(s, b)
            obase = s * (embed * batch) + bt * (ET * CHUNK)
            for et in range(n_et):
                pltpu.async_copy(
                    rowst[b].at[pl.ds(et * seg, seg)],
                    out_hbm.at[pl.ds(obase + et * (blocks_per_s * seg), seg)],
                    semw[b])
            bf = (b + NBUF - 1) % NBUF
            if wait_w:
                drain_w(bf)
            if fire_ahead:
                fire(c - f0 + NBUF - 1, bf)

        for b in range(NBUF - 1):
            fire(b, b)
        for b in range(NBUF):
            step(f0 + b, b, wait_w=(b > 0), fire_ahead=True)

        def group_body(g, _):
            c0 = f0 + g * NBUF
            for b in range(NBUF):
                step(c0 + b, b, wait_w=True, fire_ahead=True)
            return _

        lax.fori_loop(1, n_groups - 1, group_body, None)

        cL = f0 + (n_groups - 1) * NBUF
        step(cL, 0, wait_w=True, fire_ahead=True)
        for b in range(1, NBUF):
            step(cL + b, b, wait_w=True, fire_ahead=False)
        drain_w(NBUF - 1)

    return emb_kernel


def _build_detranspose(vocab, embed, n_workers):
    n_blocks = vocab // CHUNK
    vmain = n_blocks * CHUNK
    tail = vocab - vmain
    per_w = n_blocks // n_workers
    n_extra = n_blocks - per_w * n_workers
    mesh = plsc.VectorSubcoreMesh(core_axis_name="c", subcore_axis_name="s")
    num_cores = 2

    @functools.partial(
        pl.kernel,
        mesh=mesh,
        compiler_params=pltpu.CompilerParams(use_tc_tiling_on_sc=True,
                                             needs_layout_passes=False),
        out_type=jax.ShapeDtypeStruct((vocab * embed,), jnp.float32),
        scratch_types=(
            [pltpu.VMEM((embed, CHUNK), jnp.float32)] * NBUF
            + [pltpu.VMEM((CHUNK * embed,), jnp.float32)] * NBUF
            + [pltpu.VMEM((tail * embed,), jnp.float32)]
            + [pltpu.SemaphoreType.DMA] * (2 * NBUF)
        ),
    )
    def det_kernel(tt_hbm, tail_hbm, out_hbm, *rest):
        inb = rest[:NBUF]
        outb = rest[NBUF:2 * NBUF]
        tail_v = rest[2 * NBUF]
        semi = rest[2 * NBUF + 1:3 * NBUF + 1]
        semo = rest[3 * NBUF + 1:]
        wid = lax.axis_index("s") * num_cores + lax.axis_index("c")
        blk0 = wid * per_w + jnp.minimum(wid, n_extra)

        iota16 = lax.iota(jnp.int32, LANES)

        def fire_i(i, p):
            pltpu.async_copy(
                tt_hbm.at[:, pl.ds((blk0 + i) * CHUNK, CHUNK)],
                inb[p], semi[p])

        def drain_i(p):
            pltpu.make_async_copy(tt_hbm.at[:, pl.ds(0, CHUNK)],
                                  inb[p], semi[p]).wait()

        def fire_o(i, p):
            pltpu.async_copy(outb[p],
                             out_hbm.at[pl.ds((blk0 + i) * (CHUNK * embed),
                                              CHUNK * embed)],
                             semo[p])

        def drain_o(p):
            pltpu.make_async_copy(outb[p],
                                  out_hbm.at[pl.ds(0, CHUNK * embed)],
                                  semo[p]).wait()

        def transpose_blk(p):
            def d_body(d, _):
                for eb in range(embed // LANES):
                    ce = eb * LANES + lax.rem(iota16 + d, LANES)
                    for vb in range(CHUNK // LANES):
                        vidx = iota16 + vb * LANES
                        v = plsc.load_gather(inb[p], [ce, vidx])
                        plsc.store_scatter(outb[p], [vidx * embed + ce], v)
                return _

            lax.fori_loop(0, LANES, d_body, None)

        def step(i, p, wait_o, fire_next):
            drain_i(p)
            transpose_blk(p)
            fire_o(i, p)
            pf = (p + NBUF - 1) % NBUF
            if wait_o:
                drain_o(pf)
            if fire_next:
                fire_i(i + NBUF - 1, pf)

        n_det_groups = per_w // NBUF
        for p in range(NBUF - 1):
            fire_i(p, p)
        for p in range(NBUF):
            step(p, p, wait_o=(p > 0), fire_next=True)

        def group_body(g, _):
            s0 = g * NBUF
            for p in range(NBUF):
                step(s0 + p, p, wait_o=True, fire_next=True)
            return _

        lax.fori_loop(1, n_det_groups - 1, group_body, None)

        sL = (n_det_groups - 1) * NBUF
        step(sL, 0, wait_o=True, fire_next=True)
        for p in range(1, NBUF):
            step(sL + p, p, wait_o=True, fire_next=False)
        drain_o(NBUF - 1)

        @pl.when(wid < n_extra)
        def _extra():
            pltpu.sync_copy(
                tt_hbm.at[:, pl.ds((blk0 + per_w) * CHUNK, CHUNK)], inb[0])
            transpose_blk(0)
            pltpu.sync_copy(outb[0],
                            out_hbm.at[pl.ds((blk0 + per_w) * (CHUNK * embed),
                                             CHUNK * embed)])

        @pl.when(wid == n_workers - 1)
        def _tail():
            pltpu.sync_copy(tail_hbm, tail_v)
            pltpu.sync_copy(tail_v,
                            out_hbm.at[pl.ds(vmain * embed, tail * embed)])

    return det_kernel


def _build_tc_detranspose(vocab, embed):
    bw = 4096
    grid = -(-(vocab // 4) // bw)
    q_pad = grid * bw

    def body(t0, t1, t2, t3, out_ref):
        x = jnp.concatenate(
            [t0[...], t1[...], t2[...], t3[...]], axis=0)
        out_ref[...] = x.T

    in_specs = [
        pl.BlockSpec((embed, bw),
                     functools.partial(lambda j, i: (0, j * grid + i), j))
        for j in range(3)
    ] + [pl.BlockSpec((embed, bw), lambda i: (0, i))]
    fn = pl.pallas_call(
        body,
        grid=(grid,),
        in_specs=in_specs,
        out_specs=pl.BlockSpec((bw, 4 * embed), lambda i: (i, 0)),
        out_shape=jax.ShapeDtypeStruct((q_pad, 4 * embed), jnp.float32),
    )
    return fn, q_pad


def kernel(inputs, word_table, pos_table):
    batch, seq = inputs.shape
    vocab, embed = word_table.shape
    n_workers = 32

    det, q_pad = _build_tc_detranspose(vocab, embed)
    tt = word_table.T
    tt_last = jnp.pad(lax.slice(tt, (0, 3 * q_pad), (embed, vocab)),
                      ((0, 0), (0, 4 * q_pad - vocab)))
    table_packed = det(tt, tt, tt, tt_last).reshape(4 * q_pad, embed)

    idx = inputs.T.reshape(batch * seq // CHUNK, CHUNK).astype(jnp.int32)
    idx = (idx % q_pad) * 4 + idx // q_pad
    fn = _build(batch, seq, vocab, embed, n_workers)
    flat = fn(idx, table_packed, pos_table)
    x = flat.reshape(seq, embed // ET, batch // CHUNK, ET, CHUNK)
    return x.transpose(2, 4, 0, 1, 3).reshape(batch, seq, embed)


# speedup vs baseline: 1.7743x; 1.0324x over previous
import functools

import jax
import jax.numpy as jnp
from jax import lax
from jax.experimental import pallas as pl
from jax.experimental.pallas import tpu as pltpu
from jax.experimental.pallas import tpu_sc as plsc

LANES = 16
CHUNK = 128
NBUF = 4
ET = 8


def _build(batch, seq, vocab, embed, n_workers):
    total_rows = batch * seq
    per_w = total_rows // n_workers
    n_chunks = per_w // CHUNK
    n_groups = n_chunks // NBUF
    blocks_per_s = batch // CHUNK
    n_et = embed // ET
    seg = ET * CHUNK
    mesh = plsc.VectorSubcoreMesh(core_axis_name="c", subcore_axis_name="s")
    num_cores = 2

    @functools.partial(
        pl.kernel,
        mesh=mesh,
        compiler_params=pltpu.CompilerParams(use_tc_tiling_on_sc=False,
                                             needs_layout_passes=False),
        out_type=jax.ShapeDtypeStruct((total_rows * embed,), jnp.float32),
        scratch_types=[
            pltpu.VMEM((n_chunks, CHUNK), jnp.int32),
            pltpu.VMEM((NBUF, CHUNK, embed), jnp.float32),
            pltpu.VMEM((CHUNK * embed,), jnp.float32),
            pltpu.VMEM((CHUNK * embed,), jnp.float32),
            pltpu.VMEM((CHUNK * embed,), jnp.float32),
            pltpu.VMEM((CHUNK * embed,), jnp.float32),
            pltpu.VMEM((seq, embed), jnp.float32),
            pltpu.SemaphoreType.DMA,
            pltpu.SemaphoreType.DMA,
            pltpu.SemaphoreType.DMA,
            pltpu.SemaphoreType.DMA,
            pltpu.SemaphoreType.DMA,
            pltpu.SemaphoreType.DMA,
            pltpu.SemaphoreType.DMA,
            pltpu.SemaphoreType.DMA,
        ],
    )
    def emb_kernel(idx_hbm, table_hbm, pos_hbm, out_hbm,
                   idx_all, rows_v, t0, t1, t2, t3, pos_v, *sems):
        rowst = (t0, t1, t2, t3)
        semg = sems[:NBUF]
        semw = sems[NBUF:]
        wid = lax.axis_index("s") * num_cores + lax.axis_index("c")
        f0 = wid * n_chunks

        pltpu.sync_copy(pos_hbm, pos_v)
        pltpu.sync_copy(idx_hbm.at[pl.ds(f0, n_chunks)], idx_all)

        iota16 = lax.iota(jnp.int32, LANES)

        def fire(c, b):
            pltpu.async_copy(table_hbm.at[idx_all.at[c]], rows_v.at[b],
                             semg[b])

        def drain_g(b):
            pltpu.make_async_copy(table_hbm.at[idx_all.at[0]], rows_v.at[b],
                                  semg[b]).wait()

        def drain_w(b):
            pltpu.make_async_copy(rowst[b],
                                  out_hbm.at[pl.ds(0, CHUNK * embed)],
                                  semw[b]).wait()

        def transpose_add(s, b):
            sb = jnp.broadcast_to(s, (LANES,))

            def d_body(d0, _):
                for du in range(4):
                    d = d0 * 4 + du
                    for eb in range(embed // LANES):
                        ce = eb * LANES + lax.rem(iota16 + d, LANES)
                        pe = plsc.load_gather(pos_v, [sb, ce])
                        cd = ce * CHUNK
                        for rb in range(CHUNK // LANES):
                            ridx = iota16 + rb * LANES
                            v = plsc.load_gather(rows_v.at[b], [ridx, ce])
                            plsc.store_scatter(rowst[b], [cd + ridx], v + pe)
                return _

            lax.fori_loop(0, LANES // 4, d_body, None)

        def step(c, b, wait_w, fire_ahead):
            drain_g(b)
            s = c // blocks_per_s
            bt = lax.rem(c, blocks_per_s)
            transpose_add(s, b)
            obase = s * (embed * batch) + bt * (ET * CHUNK)
            for et in range(n_et):
                pltpu.async_copy(
                    rowst[b].at[pl.ds(et * seg, seg)],
                    out_hbm.at[pl.ds(obase + et * (blocks_per_s * seg), seg)],
                    semw[b])
            bf = (b + NBUF - 1) % NBUF
            if wait_w:
                drain_w(bf)
            if fire_ahead:
                fire(c - f0 + NBUF - 1, bf)

        for b in range(NBUF - 1):
            fire(b, b)
        for b in range(NBUF):
            step(f0 + b, b, wait_w=(b > 0), fire_ahead=True)

        def group_body(g, _):
            c0 = f0 + g * NBUF
            for b in range(NBUF):
                step(c0 + b, b, wait_w=True, fire_ahead=True)
            return _

        lax.fori_loop(1, n_groups - 1, group_body, None)

        cL = f0 + (n_groups - 1) * NBUF
        step(cL, 0, wait_w=True, fire_ahead=True)
        for b in range(1, NBUF):
            step(cL + b, b, wait_w=True, fire_ahead=False)
        drain_w(NBUF - 1)

    return emb_kernel


def _build_detranspose(vocab, embed, n_workers):
    n_blocks = vocab // CHUNK
    vmain = n_blocks * CHUNK
    tail = vocab - vmain
    per_w = n_blocks // n_workers
    n_extra = n_blocks - per_w * n_workers
    mesh = plsc.VectorSubcoreMesh(core_axis_name="c", subcore_axis_name="s")
    num_cores = 2

    @functools.partial(
        pl.kernel,
        mesh=mesh,
        compiler_params=pltpu.CompilerParams(use_tc_tiling_on_sc=True,
                                             needs_layout_passes=False),
        out_type=jax.ShapeDtypeStruct((vocab * embed,), jnp.float32),
        scratch_types=(
            [pltpu.VMEM((embed, CHUNK), jnp.float32)] * NBUF
            + [pltpu.VMEM((CHUNK * embed,), jnp.float32)] * NBUF
            + [pltpu.VMEM((tail * embed,), jnp.float32)]
            + [pltpu.SemaphoreType.DMA] * (2 * NBUF)
        ),
    )
    def det_kernel(tt_hbm, tail_hbm, out_hbm, *rest):
        inb = rest[:NBUF]
        outb = rest[NBUF:2 * NBUF]
        tail_v = rest[2 * NBUF]
        semi = rest[2 * NBUF + 1:3 * NBUF + 1]
        semo = rest[3 * NBUF + 1:]
        wid = lax.axis_index("s") * num_cores + lax.axis_index("c")
        blk0 = wid * per_w + jnp.minimum(wid, n_extra)

        iota16 = lax.iota(jnp.int32, LANES)

        def fire_i(i, p):
            pltpu.async_copy(
                tt_hbm.at[:, pl.ds((blk0 + i) * CHUNK, CHUNK)],
                inb[p], semi[p])

        def drain_i(p):
            pltpu.make_async_copy(tt_hbm.at[:, pl.ds(0, CHUNK)],
                                  inb[p], semi[p]).wait()

        def fire_o(i, p):
            pltpu.async_copy(outb[p],
                             out_hbm.at[pl.ds((blk0 + i) * (CHUNK * embed),
                                              CHUNK * embed)],
                             semo[p])

        def drain_o(p):
            pltpu.make_async_copy(outb[p],
                                  out_hbm.at[pl.ds(0, CHUNK * embed)],
                                  semo[p]).wait()

        def transpose_blk(p):
            def d_body(d, _):
                for eb in range(embed // LANES):
                    ce = eb * LANES + lax.rem(iota16 + d, LANES)
                    for vb in range(CHUNK // LANES):
                        vidx = iota16 + vb * LANES
                        v = plsc.load_gather(inb[p], [ce, vidx])
                        plsc.store_scatter(outb[p], [vidx * embed + ce], v)
                return _

            lax.fori_loop(0, LANES, d_body, None)

        def step(i, p, wait_o, fire_next):
            drain_i(p)
            transpose_blk(p)
            fire_o(i, p)
            pf = (p + NBUF - 1) % NBUF
            if wait_o:
                drain_o(pf)
            if fire_next:
                fire_i(i + NBUF - 1, pf)

        n_det_groups = per_w // NBUF
        for p in range(NBUF - 1):
            fire_i(p, p)
        for p in range(NBUF):
            step(p, p, wait_o=(p > 0), fire_next=True)

        def group_body(g, _):
            s0 = g * NBUF
            for p in range(NBUF):
                step(s0 + p, p, wait_o=True, fire_next=True)
            return _

        lax.fori_loop(1, n_det_groups - 1, group_body, None)

        sL = (n_det_groups - 1) * NBUF
        step(sL, 0, wait_o=True, fire_next=True)
        for p in range(1, NBUF):
            step(sL + p, p, wait_o=True, fire_next=False)
        drain_o(NBUF - 1)

        @pl.when(wid < n_extra)
        def _extra():
            pltpu.sync_copy(
                tt_hbm.at[:, pl.ds((blk0 + per_w) * CHUNK, CHUNK)], inb[0])
            transpose_blk(0)
            pltpu.sync_copy(outb[0],
                            out_hbm.at[pl.ds((blk0 + per_w) * (CHUNK * embed),
                                             CHUNK * embed)])

        @pl.when(wid == n_workers - 1)
        def _tail():
            pltpu.sync_copy(tail_hbm, tail_v)
            pltpu.sync_copy(tail_v,
                            out_hbm.at[pl.ds(vmain * embed, tail * embed)])

    return det_kernel


def _build_tc_detranspose(vocab, embed):
    bw = 8192
    grid = -(-(vocab // 4) // bw)
    q_pad = grid * bw

    def body(t0, t1, t2, t3, out_ref):
        x = jnp.concatenate(
            [t0[...], t1[...], t2[...], t3[...]], axis=0)
        out_ref[...] = x.T

    in_specs = [
        pl.BlockSpec((embed, bw),
                     functools.partial(lambda j, i: (0, j * grid + i), j))
        for j in range(3)
    ] + [pl.BlockSpec((embed, bw), lambda i: (0, i))]
    fn = pl.pallas_call(
        body,
        grid=(grid,),
        in_specs=in_specs,
        out_specs=pl.BlockSpec((bw, 4 * embed), lambda i: (i, 0)),
        out_shape=jax.ShapeDtypeStruct((q_pad, 4 * embed), jnp.float32),
    )
    return fn, q_pad


def kernel(inputs, word_table, pos_table):
    batch, seq = inputs.shape
    vocab, embed = word_table.shape
    n_workers = 32

    det, q_pad = _build_tc_detranspose(vocab, embed)
    tt = word_table.T
    tt_last = jnp.pad(lax.slice(tt, (0, 3 * q_pad), (embed, vocab)),
                      ((0, 0), (0, 4 * q_pad - vocab)))
    table_packed = det(tt, tt, tt, tt_last).reshape(4 * q_pad, embed)

    idx = inputs.T.reshape(batch * seq // CHUNK, CHUNK).astype(jnp.int32)
    idx = (idx % q_pad) * 4 + idx // q_pad
    fn = _build(batch, seq, vocab, embed, n_workers)
    flat = fn(idx, table_packed, pos_table)
    x = flat.reshape(seq, embed // ET, batch // CHUNK, ET, CHUNK)
    return x.transpose(2, 4, 0, 1, 3).reshape(batch, seq, embed)


# speedup vs baseline: 1.8331x; 1.0331x over previous
import functools

import jax
import jax.numpy as jnp
from jax import lax
from jax.experimental import pallas as pl
from jax.experimental.pallas import tpu as pltpu
from jax.experimental.pallas import tpu_sc as plsc

LANES = 16
CHUNK = 128
NBUF = 4
ET = 8


def _build(batch, seq, vocab, embed, n_workers):
    total_rows = batch * seq
    per_w = total_rows // n_workers
    n_chunks = per_w // CHUNK
    n_groups = n_chunks // NBUF
    blocks_per_s = batch // CHUNK
    n_et = embed // ET
    seg = ET * CHUNK
    mesh = plsc.VectorSubcoreMesh(core_axis_name="c", subcore_axis_name="s")
    num_cores = 2

    @functools.partial(
        pl.kernel,
        mesh=mesh,
        compiler_params=pltpu.CompilerParams(use_tc_tiling_on_sc=False,
                                             needs_layout_passes=False),
        out_type=jax.ShapeDtypeStruct((total_rows * embed,), jnp.float32),
        scratch_types=[
            pltpu.VMEM((n_chunks, CHUNK), jnp.int32),
            pltpu.VMEM((NBUF, CHUNK, embed), jnp.float32),
            pltpu.VMEM((CHUNK * embed,), jnp.float32),
            pltpu.VMEM((CHUNK * embed,), jnp.float32),
            pltpu.VMEM((CHUNK * embed,), jnp.float32),
            pltpu.VMEM((CHUNK * embed,), jnp.float32),
            pltpu.VMEM((seq, embed), jnp.float32),
            pltpu.SemaphoreType.DMA,
            pltpu.SemaphoreType.DMA,
            pltpu.SemaphoreType.DMA,
            pltpu.SemaphoreType.DMA,
            pltpu.SemaphoreType.DMA,
            pltpu.SemaphoreType.DMA,
            pltpu.SemaphoreType.DMA,
            pltpu.SemaphoreType.DMA,
        ],
    )
    def emb_kernel(idx_hbm, table_hbm, pos_hbm, out_hbm,
                   idx_all, rows_v, t0, t1, t2, t3, pos_v, *sems):
        rowst = (t0, t1, t2, t3)
        semg = sems[:NBUF]
        semw = sems[NBUF:]
        wid = lax.axis_index("s") * num_cores + lax.axis_index("c")
        f0 = wid * n_chunks

        pltpu.sync_copy(pos_hbm, pos_v)
        pltpu.sync_copy(idx_hbm.at[pl.ds(f0, n_chunks)], idx_all)

        iota16 = lax.iota(jnp.int32, LANES)

        def fire(c, b):
            pltpu.async_copy(table_hbm.at[idx_all.at[c]], rows_v.at[b],
                             semg[b])

        def drain_g(b):
            pltpu.make_async_copy(table_hbm.at[idx_all.at[0]], rows_v.at[b],
                                  semg[b]).wait()

        def drain_w(b):
            pltpu.make_async_copy(rowst[b],
                                  out_hbm.at[pl.ds(0, CHUNK * embed)],
                                  semw[b]).wait()

        def transpose_add(s, b):
            sb = jnp.broadcast_to(s, (LANES,))

            def d_body(d0, _):
                for du in range(4):
                    d = d0 * 4 + du
                    for eb in range(embed // LANES):
                        ce = eb * LANES + lax.rem(iota16 + d, LANES)
                        pe = plsc.load_gather(pos_v, [sb, ce])
                        cd = ce * CHUNK
                        for rb in range(CHUNK // LANES):
                            ridx = iota16 + rb * LANES
                            v = plsc.load_gather(rows_v.at[b], [ridx, ce])
                            plsc.store_scatter(rowst[b], [cd + ridx], v + pe)
                return _

            lax.fori_loop(0, LANES // 4, d_body, None)

        def step(c, b, wait_w, fire_ahead):
            drain_g(b)
            s = c // blocks_per_s
            bt = lax.rem(c, blocks_per_s)
            transpose_add(s, b)
            obase = s * (embed * batch) + bt * (ET * CHUNK)
            for et in range(n_et):
                pltpu.async_copy(
                    rowst[b].at[pl.ds(et * seg, seg)],
                    out_hbm.at[pl.ds(obase + et * (blocks_per_s * seg), seg)],
                    semw[b])
            bf = (b + NBUF - 1) % NBUF
            if wait_w:
                drain_w(bf)
            if fire_ahead:
                fire(c - f0 + NBUF - 1, bf)

        for b in range(NBUF - 1):
            fire(b, b)
        for b in range(NBUF):
            step(f0 + b, b, wait_w=(b > 0), fire_ahead=True)

        def group_body(g, _):
            c0 = f0 + g * NBUF
            for b in range(NBUF):
                step(c0 + b, b, wait_w=True, fire_ahead=True)
            return _

        lax.fori_loop(1, n_groups - 1, group_body, None)

        cL = f0 + (n_groups - 1) * NBUF
        step(cL, 0, wait_w=True, fire_ahead=True)
        for b in range(1, NBUF):
            step(cL + b, b, wait_w=True, fire_ahead=False)
        drain_w(NBUF - 1)

    return emb_kernel


def _build_detranspose(vocab, embed, n_workers):
    n_blocks = vocab // CHUNK
    vmain = n_blocks * CHUNK
    tail = vocab - vmain
    per_w = n_blocks // n_workers
    n_extra = n_blocks - per_w * n_workers
    mesh = plsc.VectorSubcoreMesh(core_axis_name="c", subcore_axis_name="s")
    num_cores = 2

    @functools.partial(
        pl.kernel,
        mesh=mesh,
        compiler_params=pltpu.CompilerParams(use_tc_tiling_on_sc=True,
                                             needs_layout_passes=False),
        out_type=jax.ShapeDtypeStruct((vocab * embed,), jnp.float32),
        scratch_types=(
            [pltpu.VMEM((embed, CHUNK), jnp.float32)] * NBUF
            + [pltpu.VMEM((CHUNK * embed,), jnp.float32)] * NBUF
            + [pltpu.VMEM((tail * embed,), jnp.float32)]
            + [pltpu.SemaphoreType.DMA] * (2 * NBUF)
        ),
    )
    def det_kernel(tt_hbm, tail_hbm, out_hbm, *rest):
        inb = rest[:NBUF]
        outb = rest[NBUF:2 * NBUF]
        tail_v = rest[2 * NBUF]
        semi = rest[2 * NBUF + 1:3 * NBUF + 1]
        semo = rest[3 * NBUF + 1:]
        wid = lax.axis_index("s") * num_cores + lax.axis_index("c")
        blk0 = wid * per_w + jnp.minimum(wid, n_extra)

        iota16 = lax.iota(jnp.int32, LANES)

        def fire_i(i, p):
            pltpu.async_copy(
                tt_hbm.at[:, pl.ds((blk0 + i) * CHUNK, CHUNK)],
                inb[p], semi[p])

        def drain_i(p):
            pltpu.make_async_copy(tt_hbm.at[:, pl.ds(0, CHUNK)],
                                  inb[p], semi[p]).wait()

        def fire_o(i, p):
            pltpu.async_copy(outb[p],
                             out_hbm.at[pl.ds((blk0 + i) * (CHUNK * embed),
                                              CHUNK * embed)],
                             semo[p])

        def drain_o(p):
            pltpu.make_async_copy(outb[p],
                                  out_hbm.at[pl.ds(0, CHUNK * embed)],
                                  semo[p]).wait()

        def transpose_blk(p):
            def d_body(d, _):
                for eb in range(embed // LANES):
                    ce = eb * LANES + lax.rem(iota16 + d, LANES)
                    for vb in range(CHUNK // LANES):
                        vidx = iota16 + vb * LANES
                        v = plsc.load_gather(inb[p], [ce, vidx])
                        plsc.store_scatter(outb[p], [vidx * embed + ce], v)
                return _

            lax.fori_loop(0, LANES, d_body, None)

        def step(i, p, wait_o, fire_next):
            drain_i(p)
            transpose_blk(p)
            fire_o(i, p)
            pf = (p + NBUF - 1) % NBUF
            if wait_o:
                drain_o(pf)
            if fire_next:
                fire_i(i + NBUF - 1, pf)

        n_det_groups = per_w // NBUF
        for p in range(NBUF - 1):
            fire_i(p, p)
        for p in range(NBUF):
            step(p, p, wait_o=(p > 0), fire_next=True)

        def group_body(g, _):
            s0 = g * NBUF
            for p in range(NBUF):
                step(s0 + p, p, wait_o=True, fire_next=True)
            return _

        lax.fori_loop(1, n_det_groups - 1, group_body, None)

        sL = (n_det_groups - 1) * NBUF
        step(sL, 0, wait_o=True, fire_next=True)
        for p in range(1, NBUF):
            step(sL + p, p, wait_o=True, fire_next=False)
        drain_o(NBUF - 1)

        @pl.when(wid < n_extra)
        def _extra():
            pltpu.sync_copy(
                tt_hbm.at[:, pl.ds((blk0 + per_w) * CHUNK, CHUNK)], inb[0])
            transpose_blk(0)
            pltpu.sync_copy(outb[0],
                            out_hbm.at[pl.ds((blk0 + per_w) * (CHUNK * embed),
                                             CHUNK * embed)])

        @pl.when(wid == n_workers - 1)
        def _tail():
            pltpu.sync_copy(tail_hbm, tail_v)
            pltpu.sync_copy(tail_v,
                            out_hbm.at[pl.ds(vmain * embed, tail * embed)])

    return det_kernel


def _build_tc_detranspose(vocab, embed):
    bw = 16384
    grid = -(-(vocab // 4) // bw)
    q_pad = grid * bw

    def body(t0, t1, t2, t3, out_ref):
        x = jnp.concatenate(
            [t0[...], t1[...], t2[...], t3[...]], axis=0)
        out_ref[...] = x.T

    in_specs = [
        pl.BlockSpec((embed, bw),
                     functools.partial(lambda j, i: (0, j * grid + i), j))
        for j in range(3)
    ] + [pl.BlockSpec((embed, bw), lambda i: (0, i))]
    fn = pl.pallas_call(
        body,
        grid=(grid,),
        in_specs=in_specs,
        out_specs=pl.BlockSpec((bw, 4 * embed), lambda i: (i, 0)),
        out_shape=jax.ShapeDtypeStruct((q_pad, 4 * embed), jnp.float32),
    )
    return fn, q_pad


def kernel(inputs, word_table, pos_table):
    batch, seq = inputs.shape
    vocab, embed = word_table.shape
    n_workers = 32

    det, q_pad = _build_tc_detranspose(vocab, embed)
    tt = word_table.T
    tt_last = jnp.pad(lax.slice(tt, (0, 3 * q_pad), (embed, vocab)),
                      ((0, 0), (0, 4 * q_pad - vocab)))
    table_packed = det(tt, tt, tt, tt_last).reshape(4 * q_pad, embed)

    idx = inputs.T.reshape(batch * seq // CHUNK, CHUNK).astype(jnp.int32)
    idx = (idx % q_pad) * 4 + idx // q_pad
    fn = _build(batch, seq, vocab, embed, n_workers)
    flat = fn(idx, table_packed, pos_table)
    x = flat.reshape(seq, embed // ET, batch // CHUNK, ET, CHUNK)
    return x.transpose(2, 4, 0, 1, 3).reshape(batch, seq, embed)
